# baseline (speedup 1.0000x reference)
import functools

import jax
import jax.numpy as jnp
from jax import lax
from jax.experimental import pallas as pl
from jax.experimental.pallas import tpu as pltpu
from jax.experimental.pallas import tpu_sc as plsc

VOCAB = 1000000
EMB = 16
BATCH = 16384
HIST = 50
SEQ = HIST + 2

NC = 2
NS = 16
NW = NC * NS
BPW = BATCH // NW
HPW = BPW * HIST
CH = 800
NCH = HPW // CH
NBUF = 6
NIDX = 2 * NBUF

BC2 = 4
NCH2 = BPW // BC2


@functools.lru_cache(maxsize=1)
def _build_sc_embed():
    mesh = plsc.VectorSubcoreMesh(core_axis_name="c", subcore_axis_name="s")

    @functools.partial(
        pl.kernel,
        mesh=mesh,
        out_type=jax.ShapeDtypeStruct((BATCH * SEQ, EMB), jnp.float32),
        compiler_params=pltpu.CompilerParams(use_tc_tiling_on_sc=False),
        scratch_types=[
            [pltpu.VMEM((CH,), jnp.int32) for _ in range(NIDX)],
            [pltpu.VMEM((CH,), jnp.int32) for _ in range(NIDX)],
            [pltpu.VMEM((CH, EMB), jnp.float32) for _ in range(NBUF)],
            pltpu.VMEM((BPW,), jnp.int32),
            pltpu.VMEM((BPW, EMB), jnp.float32),
            [pltpu.SemaphoreType.DMA for _ in range(NIDX)],
            [pltpu.SemaphoreType.DMA for _ in range(NBUF)],
            [pltpu.SemaphoreType.DMA for _ in range(NBUF)],
            pltpu.SemaphoreType.DMA,
        ],
    )
    def _sc_embed(idx_h, dst_h, rows_u, dst_u, rows_i, dst_i, t_h, out,
                  idx_bufs, dst_bufs, row_bufs, sdst_v, srows_v,
                  isems, gsems, ssems, ssem):
        wid = lax.axis_index("s") * NC + lax.axis_index("c")
        hbase = wid * HPW
        sbase = wid * BPW

        def load_idx(c):
            slot = c % NIDX
            a = pltpu.async_copy(idx_h.at[pl.ds(hbase + c * CH, CH)],
                                 idx_bufs[slot], isems[slot])
            b = pltpu.async_copy(dst_h.at[pl.ds(hbase + c * CH, CH)],
                                 dst_bufs[slot], isems[slot])
            return (a, b)

        def gather(c):
            return pltpu.async_copy(t_h.at[idx_bufs[c % NIDX]],
                                    row_bufs[c % NBUF], gsems[c % NBUF])

        def scatter(c):
            return pltpu.async_copy(row_bufs[c % NBUF],
                                    out.at[dst_bufs[c % NIDX]],
                                    ssems[c % NBUF])

        i_pend = {}
        g_pend = {}
        s_pend = {}
        for c in range(min(NBUF, NCH)):
            i_pend[c % NIDX] = load_idx(c)

        for c in range(NCH):
            if c >= NBUF:
                s_pend.pop((c - NBUF) % NBUF).wait()
            if c + NBUF < NCH:
                i_pend[(c + NBUF) % NIDX] = load_idx(c + NBUF)
            a, b = i_pend.pop(c % NIDX)
            a.wait()
            b.wait()
            g_pend[c % NBUF] = gather(c)
            if c >= 1:
                g_pend.pop((c - 1) % NBUF).wait()
                s_pend[(c - 1) % NBUF] = scatter(c - 1)

        g_pend.pop((NCH - 1) % NBUF).wait()
        s_pend[(NCH - 1) % NBUF] = scatter(NCH - 1)

        def small_scatter(rows_hbm, dst_hbm):
            pltpu.sync_copy(rows_hbm.at[pl.ds(sbase, BPW)], srows_v)
            pltpu.sync_copy(dst_hbm.at[pl.ds(sbase, BPW)], sdst_v)
            pltpu.async_copy(srows_v, out.at[sdst_v], ssem).wait()

        small_scatter(rows_u, dst_u)
        small_scatter(rows_i, dst_i)

        for slot in list(s_pend):
            s_pend.pop(slot).wait()

    return _sc_embed


@functools.lru_cache(maxsize=1)
def _build_sc_format():
    mesh = plsc.VectorSubcoreMesh(core_axis_name="c", subcore_axis_name="s")
    chunk_elems = BC2 * SEQ * EMB

    @functools.partial(
        pl.kernel,
        mesh=mesh,
        out_type=jax.ShapeDtypeStruct((BATCH, SEQ, EMB), jnp.float32),
        compiler_params=pltpu.CompilerParams(use_tc_tiling_on_sc=True),
        scratch_types=[
            [pltpu.VMEM((chunk_elems,), jnp.float32) for _ in range(2)],
            [pltpu.VMEM((BC2, SEQ, EMB), jnp.float32) for _ in range(2)],
            [pltpu.SemaphoreType.DMA for _ in range(2)],
            [pltpu.SemaphoreType.DMA for _ in range(2)],
        ],
    )
    def _sc_format(rows1d, out, lin_bufs, asm_bufs, isems, osems):
        wid = lax.axis_index("s") * NC + lax.axis_index("c")
        base_b = wid * BPW

        def load(k, slot):
            off = (base_b + k * BC2) * SEQ * EMB
            return pltpu.async_copy(rows1d.at[pl.ds(off, chunk_elems)],
                                    lin_bufs[slot], isems[slot])

        def assemble(slot):
            lin = lin_bufs[slot]
            asm = asm_bufs[slot]
            for b in range(BC2):
                for s in range(SEQ):
                    asm[b, s, :] = lin[pl.ds((b * SEQ + s) * EMB, EMB)]

        def store(k, slot):
            return pltpu.async_copy(asm_bufs[slot],
                                    out.at[pl.ds(base_b + k * BC2, BC2)],
                                    osems[slot])

        def wait_load(slot):
            pltpu.make_async_copy(rows1d.at[pl.ds(0, chunk_elems)],
                                  lin_bufs[slot], isems[slot]).wait()

        def wait_store(slot):
            pltpu.make_async_copy(asm_bufs[slot],
                                  out.at[pl.ds(base_b, BC2)],
                                  osems[slot]).wait()

        def body(j, carry):
            for slot in range(2):
                k = j * 2 + slot
                wait_load(slot)

                @pl.when(j >= 1)
                def _():
                    wait_store(slot)

                assemble(slot)

                @pl.when(k + 2 < NCH2)
                def _():
                    load(k + 2, slot)

                store(k, slot)
            return carry

        load(0, 0)
        load(1, 1)
        lax.fori_loop(0, NCH2 // 2, body, 0, unroll=False)
        for slot in range(2):
            wait_store(slot)

    return _sc_format


def kernel(hist_item, user_id, item_id, T_hist, T_user, T_item):
    idx_h = hist_item.astype(jnp.int32).reshape(-1)
    rows_u = jnp.take(T_user, user_id.reshape(-1), axis=0)
    rows_i = jnp.take(T_item, item_id.reshape(-1), axis=0)
    row0 = jnp.arange(BATCH, dtype=jnp.int32) * SEQ
    dst_h = (row0[:, None]
             + jnp.arange(HIST, dtype=jnp.int32)[None, :]).reshape(-1)
    dst_u = row0 + HIST
    dst_i = row0 + HIST + 1
    rows = _build_sc_embed()(idx_h, dst_h, rows_u, dst_u, rows_i, dst_i,
                             T_hist)
    return _build_sc_format()(rows.reshape(-1))
